# baseline (speedup 1.0000x reference)
import jax
import jax.numpy as jnp
from jax import lax
from jax.experimental import pallas as pl
from jax.experimental.pallas import tpu as pltpu
from jax.experimental.pallas import tpu_sc as plsc

B = 16384
F = 26
H = 64
N = B * F
PRIME = 2147483647
MASK31 = 0x7FFFFFFF
MASK22 = 4194303
NW = 32
IDS_PER_TILE = N // NW
CHUNK = 128
NCHUNK = IDS_PER_TILE // CHUNK
NGRP = NCHUNK // 4
CHUNK_OUT = CHUNK * H
NXV = CHUNK // 16


def _u32(v):
    return jnp.uint32(v)


def _body(x_hbm, tab_hbm, a0_hbm, a1_hbm, b_hbm, out_hbm,
          xall, x0b, x1b, posb,
          idx0, idx1, idx2, idx3, g0, g1, g2, g3,
          a0v, a1v, bv,
          sg0, sg1, sg2, sg3, sw0, sw1, sw2, sw3):
    c = lax.axis_index("c")
    s = lax.axis_index("s")
    wid = s * jnp.int32(2) + c
    tile_xbase = wid * jnp.int32(IDS_PER_TILE)
    tile_obase = tile_xbase * jnp.int32(H)

    pltpu.sync_copy(x_hbm.at[pl.ds(tile_xbase, IDS_PER_TILE)], xall)
    pltpu.sync_copy(a0_hbm, a0v)
    pltpu.sync_copy(a1_hbm, a1v)
    pltpu.sync_copy(b_hbm, bv)

    for iv in range(NXV):
        lanes = jnp.int32(iv * 16) + lax.iota(jnp.int32, 16)
        posb[pl.ds(iv * 16, 16)] = lax.shift_left(lanes, jnp.int32(6))

    def compute_idx(gi, idxb, gb, sg):
        cb = gi * jnp.int32(CHUNK)
        for iv in range(NXV):
            xu = plsc.bitcast(xall[pl.ds(cb + jnp.int32(iv * 16), 16)], jnp.uint32)
            x0 = xu & _u32(0xFFFF)
            x1 = lax.shift_right_logical(xu, _u32(16))
            pos = posb[pl.ds(iv * 16, 16)]

            def j_body(jj, carry):
                for uu in range(4):
                    j = jj * jnp.int32(4) + jnp.int32(uu)
                    a0s = a0v[j]
                    a1s = a1v[j]
                    bs = bv[j]
                    lo = x0 * a0s
                    mid = x1 * a0s + x0 * a1s
                    hi = x1 * a1s
                    m1 = lax.shift_right_logical(mid, _u32(15))
                    m0 = mid & _u32(0x7FFF)
                    l1 = lax.shift_right_logical(lo, _u32(31))
                    l0 = lo & _u32(MASK31)
                    u = lax.shift_left(hi, _u32(1)) + m1 + l1
                    t = u + lax.shift_left(m0, _u32(16))
                    t = lax.shift_right_logical(t, _u32(31)) + (t & _u32(MASK31))
                    t = t + l0
                    t = lax.shift_right_logical(t, _u32(31)) + (t & _u32(MASK31))
                    t = t + bs
                    t = lax.shift_right_logical(t, _u32(31)) + (t & _u32(MASK31))
                    t = t + lax.shift_right_logical(t + _u32(1), _u32(31))
                    h = plsc.bitcast(t & _u32(MASK22), jnp.int32)
                    plsc.store_scatter(idxb, [pos + j], h)
                return carry

            lax.fori_loop(jnp.int32(0), jnp.int32(H // 4), j_body, jnp.int32(0))
            off = jnp.int32(iv * 1024)
            pltpu.async_copy(tab_hbm.at[idxb.at[pl.ds(off, 1024)]],
                             gb.at[pl.ds(off, 1024)], sg)

    def start_writeout(gb, gi, sem_w):
        pltpu.async_copy(
            gb, out_hbm.at[pl.ds(tile_obase + gi * jnp.int32(CHUNK_OUT), CHUNK_OUT)],
            sem_w)

    def wait_gather(idxb, gb, sem_g):
        for _ in range(NXV):
            pltpu.make_async_copy(tab_hbm.at[idxb.at[pl.ds(0, 1024)]],
                                  gb.at[pl.ds(0, 1024)], sem_g).wait()

    def drain_writeout(gb, sem_w):
        pltpu.make_async_copy(gb, out_hbm.at[pl.ds(0, CHUNK_OUT)], sem_w).wait()

    slots = [(idx0, g0, sg0, sw0), (idx1, g1, sg1, sw1),
             (idx2, g2, sg2, sw2), (idx3, g3, sg3, sw3)]

    def group_body(k, carry):
        for r in range(4):
            idxb, gb, sg, sw = slots[r]
            idxp, gp, sgp, swp = slots[(r + 2) % 4]
            g = k * jnp.int32(4) + jnp.int32(r)

            @pl.when(k > jnp.int32(0))
            def _():
                drain_writeout(gb, sw)

            compute_idx(g, idxb, gb, sg)

            @pl.when(g >= jnp.int32(2))
            def _():
                wait_gather(idxp, gp, sgp)
                start_writeout(gp, g - jnp.int32(2), swp)
        return carry

    lax.fori_loop(jnp.int32(0), jnp.int32(NGRP), group_body, jnp.int32(0))

    wait_gather(idx2, g2, sg2)
    start_writeout(g2, jnp.int32(NCHUNK - 2), sw2)
    wait_gather(idx3, g3, sg3)
    start_writeout(g3, jnp.int32(NCHUNK - 1), sw3)
    drain_writeout(g0, sw0)
    drain_writeout(g1, sw1)
    drain_writeout(g2, sw2)
    drain_writeout(g3, sw3)


@jax.jit
def _sc_lookup(x32, table, a0b, a1b, bb):
    mesh = plsc.VectorSubcoreMesh(core_axis_name="c", subcore_axis_name="s")
    return pl.kernel(
        _body,
        out_type=jax.ShapeDtypeStruct((N * H,), jnp.float32),
        mesh=mesh,
        compiler_params=pltpu.CompilerParams(needs_layout_passes=False, disable_bounds_checks=True),
        scratch_types=[
            pltpu.VMEM((IDS_PER_TILE,), jnp.int32),
            pltpu.VMEM((CHUNK,), jnp.uint32),
            pltpu.VMEM((CHUNK,), jnp.uint32),
            pltpu.VMEM((CHUNK,), jnp.int32),
            pltpu.VMEM((CHUNK_OUT,), jnp.int32),
            pltpu.VMEM((CHUNK_OUT,), jnp.int32),
            pltpu.VMEM((CHUNK_OUT,), jnp.int32),
            pltpu.VMEM((CHUNK_OUT,), jnp.int32),
            pltpu.VMEM((CHUNK_OUT,), jnp.float32),
            pltpu.VMEM((CHUNK_OUT,), jnp.float32),
            pltpu.VMEM((CHUNK_OUT,), jnp.float32),
            pltpu.VMEM((CHUNK_OUT,), jnp.float32),
            pltpu.VMEM((H, 16), jnp.uint32),
            pltpu.VMEM((H, 16), jnp.uint32),
            pltpu.VMEM((H, 16), jnp.uint32),
            pltpu.SemaphoreType.DMA,
            pltpu.SemaphoreType.DMA,
            pltpu.SemaphoreType.DMA,
            pltpu.SemaphoreType.DMA,
            pltpu.SemaphoreType.DMA,
            pltpu.SemaphoreType.DMA,
            pltpu.SemaphoreType.DMA,
            pltpu.SemaphoreType.DMA,
        ],
    )(x32, table, a0b, a1b, bb)


def kernel(x, table, a, b):
    x32 = x.reshape(-1).astype(jnp.int32)
    a0 = jnp.broadcast_to((a & 0xFFFF).astype(jnp.uint32)[:, None], (H, 16))
    a1 = jnp.broadcast_to((a >> 16).astype(jnp.uint32)[:, None], (H, 16))
    bb = jnp.broadcast_to(b.astype(jnp.uint32)[:, None], (H, 16))
    out = _sc_lookup(x32, table, a0, a1, bb)
    return out.reshape(B, F, H)
